# Initial kernel scaffold; baseline (speedup 1.0000x reference)
import functools

import jax
import jax.numpy as jnp
from jax import lax
from jax.experimental import pallas as pl
from jax.experimental.pallas import tpu as pltpu

E = 160000
EMB = 128
NR = 6
NS = 7
NB = 8

BLK = 1280


def _edge_transform_body(m_ref, rbf_ref, sbf_ref, W_rbf_ref, W_ji_ref, b_ji_ref,
                         W_kj_ref, b_kj_ref, W_sbf_ref,
                         x_ji_ref, x_kj_ref, sbf_p_ref):
    m = m_ref[...]
    rbf_p = rbf_ref[...] @ W_rbf_ref[...]
    x_ji_ref[...] = m @ W_ji_ref[...] + b_ji_ref[...][None, :]
    x_kj_ref[...] = (m @ W_kj_ref[...] + b_kj_ref[...][None, :]) * rbf_p
    sbf_p_ref[...] = sbf_ref[...] @ W_sbf_ref[...]


def _edge_transform(m, rbf, sbf, W_rbf, W_ji, b_ji, W_kj, b_kj, W_sbf):
    nblk = E // BLK
    row = lambda i: (i, 0)
    full = lambda i: (0, 0)
    return pl.pallas_call(
        _edge_transform_body,
        grid=(nblk,),
        in_specs=[
            pl.BlockSpec((BLK, EMB), row),
            pl.BlockSpec((BLK, NR), row),
            pl.BlockSpec((BLK, NR * NS), row),
            pl.BlockSpec((NR, EMB), full),
            pl.BlockSpec((EMB, EMB), full),
            pl.BlockSpec((EMB,), lambda i: (0,)),
            pl.BlockSpec((EMB, EMB), full),
            pl.BlockSpec((EMB,), lambda i: (0,)),
            pl.BlockSpec((NR * NS, NB), full),
        ],
        out_specs=[
            pl.BlockSpec((BLK, EMB), row),
            pl.BlockSpec((BLK, EMB), row),
            pl.BlockSpec((BLK, NB), row),
        ],
        out_shape=[
            jax.ShapeDtypeStruct((E, EMB), jnp.float32),
            jax.ShapeDtypeStruct((E, EMB), jnp.float32),
            jax.ShapeDtypeStruct((E, NB), jnp.float32),
        ],
    )(m, rbf, sbf, W_rbf, W_ji, b_ji, W_kj, b_kj, W_sbf)


def _bilinear_body(xk_ref, sbf_p_ref, Wb_ref, msg_ref):
    xk = xk_ref[...]
    sbf_p = sbf_p_ref[...]
    acc = jnp.zeros((BLK, EMB), jnp.float32)
    for l in range(NB):
        t = jax.lax.dot_general(
            xk, Wb_ref[:, l * EMB:(l + 1) * EMB],
            (((1,), (0,)), ((), ())), preferred_element_type=jnp.float32)
        acc = acc + sbf_p[:, l:l + 1] * t
    msg_ref[...] = acc


def _bilinear(xk, sbf_p, Wb):
    nblk = E // BLK
    return pl.pallas_call(
        _bilinear_body,
        grid=(nblk,),
        in_specs=[
            pl.BlockSpec((BLK, EMB), lambda i: (i, 0)),
            pl.BlockSpec((BLK, NB), lambda i: (i, 0)),
            pl.BlockSpec((EMB, NB * EMB), lambda i: (0, 0)),
        ],
        out_specs=pl.BlockSpec((BLK, EMB), lambda i: (i, 0)),
        out_shape=jax.ShapeDtypeStruct((E, EMB), jnp.float32),
    )(xk, sbf_p, Wb)


def _residual_body(mu_ref, xji_ref, m_ref,
                   w1_ref, b1_ref, w2_ref, b2_ref, wf_ref, bf_ref,
                   a1_ref, ab1_ref, a2_ref, ab2_ref, a3_ref, ab3_ref,
                   a4_ref, ab4_ref, out_ref):
    f32 = jnp.float32
    mm = lambda a, b: jax.lax.dot_general(a, b, (((1,), (0,)), ((), ())),
                                          preferred_element_type=f32)
    h = mu_ref[...] + xji_ref[...]
    h = h + mm(mm(h, w1_ref[...]) + b1_ref[...][None, :], w2_ref[...]) + b2_ref[...][None, :]
    h = mm(h, wf_ref[...]) + bf_ref[...][None, :]
    out = m_ref[...] + h
    out = out + mm(mm(out, a1_ref[...]) + ab1_ref[...][None, :], a2_ref[...]) + ab2_ref[...][None, :]
    out = out + mm(mm(out, a3_ref[...]) + ab3_ref[...][None, :], a4_ref[...]) + ab4_ref[...][None, :]
    out_ref[...] = out


def _residual_stack(m_update, x_ji, m, rb1_W1, rb1_b1, rb1_W2, rb1_b2,
                    W_final, b_final, ra1_W1, ra1_b1, ra1_W2, ra1_b2,
                    ra2_W1, ra2_b1, ra2_W2, ra2_b2):
    nblk = E // BLK
    row = lambda i: (i, 0)
    full = lambda i: (0, 0)
    vec = lambda i: (0,)
    wspec = []
    for _ in range(8):
        wspec += [pl.BlockSpec((EMB, EMB), full), pl.BlockSpec((EMB,), vec)]
    return pl.pallas_call(
        _residual_body,
        grid=(nblk,),
        in_specs=[pl.BlockSpec((BLK, EMB), row)] * 3 + wspec,
        out_specs=pl.BlockSpec((BLK, EMB), row),
        out_shape=jax.ShapeDtypeStruct((E, EMB), jnp.float32),
    )(m_update, x_ji, m, rb1_W1, rb1_b1, rb1_W2, rb1_b2, W_final, b_final,
      ra1_W1, ra1_b1, ra1_W2, ra1_b2, ra2_W1, ra2_b1, ra2_W2, ra2_b2)


def kernel(m, rbf, sbf, lg_edge_index, W_rbf, W_sbf, W_ji, b_ji, W_kj, b_kj,
           W_bilin, rb1_W1, rb1_b1, rb1_W2, rb1_b2, W_final, b_final,
           ra1_W1, ra1_b1, ra1_W2, ra1_b2, ra2_W1, ra2_b1, ra2_W2, ra2_b2):
    src = lg_edge_index[0]
    dst = lg_edge_index[1]

    x_ji, x_kj, sbf_p = _edge_transform(m, rbf, sbf, W_rbf, W_ji, b_ji,
                                        W_kj, b_kj, W_sbf)

    xk = jnp.take(x_kj, src, axis=0)

    Wb = jnp.reshape(jnp.transpose(W_bilin, (2, 1, 0)), (EMB, NB * EMB))
    msg = _bilinear(xk, sbf_p, Wb)

    m_update = jax.ops.segment_sum(msg, dst, num_segments=E)

    return _residual_stack(m_update, x_ji, m, rb1_W1, rb1_b1, rb1_W2, rb1_b2,
                           W_final, b_final, ra1_W1, ra1_b1, ra1_W2, ra1_b2,
                           ra2_W1, ra2_b1, ra2_W2, ra2_b2)


# speedup vs baseline: 1.4595x; 1.4595x over previous
import functools

import jax
import jax.numpy as jnp
from jax import lax
from jax.experimental import pallas as pl
from jax.experimental.pallas import tpu as pltpu

E = 160000
EMB = 128
NR = 6
NS = 7
NB = 8

BLK = 1280


def _edge_transform_body(m_ref, rbf_ref, sbf_ref, W_rbf_ref, W_ji_ref, b_ji_ref,
                         W_kj_ref, b_kj_ref, W_sbf_ref,
                         x_ji_ref, x_kj_ref, sbf_p_ref):
    m = m_ref[...]
    rbf_p = rbf_ref[...] @ W_rbf_ref[...]
    x_ji_ref[...] = m @ W_ji_ref[...] + b_ji_ref[...][None, :]
    x_kj_ref[...] = (m @ W_kj_ref[...] + b_kj_ref[...][None, :]) * rbf_p
    sbf_p_ref[...] = sbf_ref[...] @ W_sbf_ref[...]


def _edge_transform(m, rbf, sbf, W_rbf, W_ji, b_ji, W_kj, b_kj, W_sbf):
    nblk = E // BLK
    row = lambda i: (i, 0)
    full = lambda i: (0, 0)
    return pl.pallas_call(
        _edge_transform_body,
        grid=(nblk,),
        in_specs=[
            pl.BlockSpec((BLK, EMB), row),
            pl.BlockSpec((BLK, NR), row),
            pl.BlockSpec((BLK, NR * NS), row),
            pl.BlockSpec((NR, EMB), full),
            pl.BlockSpec((EMB, EMB), full),
            pl.BlockSpec((EMB,), lambda i: (0,)),
            pl.BlockSpec((EMB, EMB), full),
            pl.BlockSpec((EMB,), lambda i: (0,)),
            pl.BlockSpec((NR * NS, NB), full),
        ],
        out_specs=[
            pl.BlockSpec((BLK, EMB), row),
            pl.BlockSpec((BLK, EMB), row),
            pl.BlockSpec((BLK, NB), row),
        ],
        out_shape=[
            jax.ShapeDtypeStruct((E, EMB), jnp.float32),
            jax.ShapeDtypeStruct((E, EMB), jnp.float32),
            jax.ShapeDtypeStruct((E, NB), jnp.float32),
        ],
    )(m, rbf, sbf, W_rbf, W_ji, b_ji, W_kj, b_kj, W_sbf)


def _bilinear_body(xk_ref, sbf_p_ref, Wb_ref, msg_ref):
    xk = xk_ref[...]
    sbf_p = sbf_p_ref[...]
    acc = jnp.zeros((BLK, EMB), jnp.float32)
    for l in range(NB):
        t = jax.lax.dot_general(
            xk, Wb_ref[:, l * EMB:(l + 1) * EMB],
            (((1,), (0,)), ((), ())), preferred_element_type=jnp.float32)
        acc = acc + sbf_p[:, l:l + 1] * t
    msg_ref[...] = acc


def _bilinear(xk, sbf_p, Wb):
    nblk = E // BLK
    return pl.pallas_call(
        _bilinear_body,
        grid=(nblk,),
        in_specs=[
            pl.BlockSpec((BLK, EMB), lambda i: (i, 0)),
            pl.BlockSpec((BLK, NB), lambda i: (i, 0)),
            pl.BlockSpec((EMB, NB * EMB), lambda i: (0, 0)),
        ],
        out_specs=pl.BlockSpec((BLK, EMB), lambda i: (i, 0)),
        out_shape=jax.ShapeDtypeStruct((E, EMB), jnp.float32),
    )(xk, sbf_p, Wb)


def _residual_body(mu_ref, xji_ref, m_ref,
                   w1_ref, b1_ref, w2_ref, b2_ref, wf_ref, bf_ref,
                   a1_ref, ab1_ref, a2_ref, ab2_ref, a3_ref, ab3_ref,
                   a4_ref, ab4_ref, out_ref):
    f32 = jnp.float32
    mm = lambda a, b: jax.lax.dot_general(a, b, (((1,), (0,)), ((), ())),
                                          preferred_element_type=f32)
    h = mu_ref[...] + xji_ref[...]
    h = h + mm(mm(h, w1_ref[...]) + b1_ref[...][None, :], w2_ref[...]) + b2_ref[...][None, :]
    h = mm(h, wf_ref[...]) + bf_ref[...][None, :]
    out = m_ref[...] + h
    out = out + mm(mm(out, a1_ref[...]) + ab1_ref[...][None, :], a2_ref[...]) + ab2_ref[...][None, :]
    out = out + mm(mm(out, a3_ref[...]) + ab3_ref[...][None, :], a4_ref[...]) + ab4_ref[...][None, :]
    out_ref[...] = out


def _residual_stack(m_update, x_ji, m, rb1_W1, rb1_b1, rb1_W2, rb1_b2,
                    W_final, b_final, ra1_W1, ra1_b1, ra1_W2, ra1_b2,
                    ra2_W1, ra2_b1, ra2_W2, ra2_b2):
    nblk = E // BLK
    row = lambda i: (i, 0)
    full = lambda i: (0, 0)
    vec = lambda i: (0,)
    wspec = []
    for _ in range(7):
        wspec += [pl.BlockSpec((EMB, EMB), full), pl.BlockSpec((EMB,), vec)]
    return pl.pallas_call(
        _residual_body,
        grid=(nblk,),
        in_specs=[pl.BlockSpec((BLK, EMB), row)] * 3 + wspec,
        out_specs=pl.BlockSpec((BLK, EMB), row),
        out_shape=jax.ShapeDtypeStruct((E, EMB), jnp.float32),
    )(m_update, x_ji, m, rb1_W1, rb1_b1, rb1_W2, rb1_b2, W_final, b_final,
      ra1_W1, ra1_b1, ra1_W2, ra1_b2, ra2_W1, ra2_b1, ra2_W2, ra2_b2)


def kernel(m, rbf, sbf, lg_edge_index, W_rbf, W_sbf, W_ji, b_ji, W_kj, b_kj,
           W_bilin, rb1_W1, rb1_b1, rb1_W2, rb1_b2, W_final, b_final,
           ra1_W1, ra1_b1, ra1_W2, ra1_b2, ra2_W1, ra2_b1, ra2_W2, ra2_b2):
    src = lg_edge_index[0]
    dst = lg_edge_index[1]

    x_ji, x_kj, sbf_p = _edge_transform(m, rbf, sbf, W_rbf, W_ji, b_ji,
                                        W_kj, b_kj, W_sbf)

    xk = jnp.take(x_kj, src, axis=0)

    Wb = jnp.reshape(jnp.transpose(W_bilin, (2, 1, 0)), (EMB, NB * EMB))
    msg = _bilinear(xk, sbf_p, Wb)

    m_update = jax.ops.segment_sum(msg, dst, num_segments=E)

    return _residual_stack(m_update, x_ji, m, rb1_W1, rb1_b1, rb1_W2, rb1_b2,
                           W_final, b_final, ra1_W1, ra1_b1, ra1_W2, ra1_b2,
                           ra2_W1, ra2_b1, ra2_W2, ra2_b2)


# speedup vs baseline: 1.5529x; 1.0640x over previous
import functools

import jax
import jax.numpy as jnp
from jax import lax
from jax.experimental import pallas as pl
from jax.experimental.pallas import tpu as pltpu
from jax.experimental.pallas import tpu_sc as plsc

E = 160000
EMB = 128
NR = 6
NS = 7
NB = 8

BLK = 1280


def _edge_transform_body(m_ref, rbf_ref, sbf_ref, W_rbf_ref, W_ji_ref, b_ji_ref,
                         W_kj_ref, b_kj_ref, W_sbf_ref,
                         x_ji_ref, x_kj_ref, sbf_p_ref):
    m = m_ref[...]
    rbf_p = rbf_ref[...] @ W_rbf_ref[...]
    x_ji_ref[...] = m @ W_ji_ref[...] + b_ji_ref[...][None, :]
    x_kj_ref[...] = (m @ W_kj_ref[...] + b_kj_ref[...][None, :]) * rbf_p
    sbf_p_ref[...] = sbf_ref[...] @ W_sbf_ref[...]


def _edge_transform(m, rbf, sbf, W_rbf, W_ji, b_ji, W_kj, b_kj, W_sbf):
    nblk = E // BLK
    row = lambda i: (i, 0)
    full = lambda i: (0, 0)
    return pl.pallas_call(
        _edge_transform_body,
        grid=(nblk,),
        in_specs=[
            pl.BlockSpec((BLK, EMB), row),
            pl.BlockSpec((BLK, NR), row),
            pl.BlockSpec((BLK, NR * NS), row),
            pl.BlockSpec((NR, EMB), full),
            pl.BlockSpec((EMB, EMB), full),
            pl.BlockSpec((EMB,), lambda i: (0,)),
            pl.BlockSpec((EMB, EMB), full),
            pl.BlockSpec((EMB,), lambda i: (0,)),
            pl.BlockSpec((NR * NS, NB), full),
        ],
        out_specs=[
            pl.BlockSpec((BLK, EMB), row),
            pl.BlockSpec((BLK, EMB), row),
            pl.BlockSpec((BLK, NB), row),
        ],
        out_shape=[
            jax.ShapeDtypeStruct((E, EMB), jnp.float32),
            jax.ShapeDtypeStruct((E, EMB), jnp.float32),
            jax.ShapeDtypeStruct((E, NB), jnp.float32),
        ],
    )(m, rbf, sbf, W_rbf, W_ji, b_ji, W_kj, b_kj, W_sbf)


def _bilinear_body(xk_ref, sbf_p_ref, Wb_ref, msg_ref):
    xk = xk_ref[...]
    sbf_p = sbf_p_ref[...]
    acc = jnp.zeros((BLK, EMB), jnp.float32)
    for l in range(NB):
        t = jax.lax.dot_general(
            xk, Wb_ref[:, l * EMB:(l + 1) * EMB],
            (((1,), (0,)), ((), ())), preferred_element_type=jnp.float32)
        acc = acc + sbf_p[:, l:l + 1] * t
    msg_ref[...] = acc


def _bilinear(xk, sbf_p, Wb):
    nblk = E // BLK
    return pl.pallas_call(
        _bilinear_body,
        grid=(nblk,),
        in_specs=[
            pl.BlockSpec((BLK, EMB), lambda i: (i, 0)),
            pl.BlockSpec((BLK, NB), lambda i: (i, 0)),
            pl.BlockSpec((EMB, NB * EMB), lambda i: (0, 0)),
        ],
        out_specs=pl.BlockSpec((BLK, EMB), lambda i: (i, 0)),
        out_shape=jax.ShapeDtypeStruct((E, EMB), jnp.float32),
    )(xk, sbf_p, Wb)


def _residual_body(mu_ref, xji_ref, m_ref,
                   w1_ref, b1_ref, w2_ref, b2_ref, wf_ref, bf_ref,
                   a1_ref, ab1_ref, a2_ref, ab2_ref, a3_ref, ab3_ref,
                   a4_ref, ab4_ref, out_ref):
    f32 = jnp.float32
    mm = lambda a, b: jax.lax.dot_general(a, b, (((1,), (0,)), ((), ())),
                                          preferred_element_type=f32)
    h = mu_ref[...] + xji_ref[...]
    h = h + mm(mm(h, w1_ref[...]) + b1_ref[...][None, :], w2_ref[...]) + b2_ref[...][None, :]
    h = mm(h, wf_ref[...]) + bf_ref[...][None, :]
    out = m_ref[...] + h
    out = out + mm(mm(out, a1_ref[...]) + ab1_ref[...][None, :], a2_ref[...]) + ab2_ref[...][None, :]
    out = out + mm(mm(out, a3_ref[...]) + ab3_ref[...][None, :], a4_ref[...]) + ab4_ref[...][None, :]
    out_ref[...] = out


def _residual_stack(m_update, x_ji, m, rb1_W1, rb1_b1, rb1_W2, rb1_b2,
                    W_final, b_final, ra1_W1, ra1_b1, ra1_W2, ra1_b2,
                    ra2_W1, ra2_b1, ra2_W2, ra2_b2):
    nblk = E // BLK
    row = lambda i: (i, 0)
    full = lambda i: (0, 0)
    vec = lambda i: (0,)
    wspec = []
    for _ in range(7):
        wspec += [pl.BlockSpec((EMB, EMB), full), pl.BlockSpec((EMB,), vec)]
    return pl.pallas_call(
        _residual_body,
        grid=(nblk,),
        in_specs=[pl.BlockSpec((BLK, EMB), row)] * 3 + wspec,
        out_specs=pl.BlockSpec((BLK, EMB), row),
        out_shape=jax.ShapeDtypeStruct((E, EMB), jnp.float32),
    )(m_update, x_ji, m, rb1_W1, rb1_b1, rb1_W2, rb1_b2, W_final, b_final,
      ra1_W1, ra1_b1, ra1_W2, ra1_b2, ra2_W1, ra2_b1, ra2_W2, ra2_b2)


_GCHUNK = 200
_NW = 32


def _sc_gather(src, x_kj):
    nchunks_per_w = E // (_GCHUNK * _NW)
    mesh = plsc.VectorSubcoreMesh(core_axis_name="c", subcore_axis_name="s")

    @functools.partial(
        pl.kernel, mesh=mesh,
        out_type=jax.ShapeDtypeStruct((E, EMB), jnp.float32),
        scratch_types=[
            pltpu.VMEM((_GCHUNK,), jnp.int32),
            pltpu.VMEM((_GCHUNK, EMB), jnp.float32),
            pltpu.SemaphoreType.DMA,
        ],
    )
    def gather_k(src_hbm, xkj_hbm, out_hbm, idx_v, rows_v, sem):
        wid = lax.axis_index("s") * 2 + lax.axis_index("c")

        def body(j, carry):
            base = (wid + _NW * j) * _GCHUNK
            pltpu.sync_copy(src_hbm.at[pl.ds(base, _GCHUNK)], idx_v)
            pltpu.async_copy(xkj_hbm.at[idx_v], rows_v, sem).wait()
            pltpu.sync_copy(rows_v, out_hbm.at[pl.ds(base, _GCHUNK)])
            return carry

        lax.fori_loop(0, nchunks_per_w, body, 0)

    return gather_k(src, x_kj)


def kernel(m, rbf, sbf, lg_edge_index, W_rbf, W_sbf, W_ji, b_ji, W_kj, b_kj,
           W_bilin, rb1_W1, rb1_b1, rb1_W2, rb1_b2, W_final, b_final,
           ra1_W1, ra1_b1, ra1_W2, ra1_b2, ra2_W1, ra2_b1, ra2_W2, ra2_b2):
    src = lg_edge_index[0]
    dst = lg_edge_index[1]

    x_ji, x_kj, sbf_p = _edge_transform(m, rbf, sbf, W_rbf, W_ji, b_ji,
                                        W_kj, b_kj, W_sbf)

    xk = _sc_gather(src, x_kj)

    Wb = jnp.reshape(jnp.transpose(W_bilin, (2, 1, 0)), (EMB, NB * EMB))
    msg = _bilinear(xk, sbf_p, Wb)

    m_update = jax.ops.segment_sum(msg, dst, num_segments=E)

    return _residual_stack(m_update, x_ji, m, rb1_W1, rb1_b1, rb1_W2, rb1_b2,
                           W_final, b_final, ra1_W1, ra1_b1, ra1_W2, ra1_b2,
                           ra2_W1, ra2_b1, ra2_W2, ra2_b2)
